# Initial kernel scaffold; baseline (speedup 1.0000x reference)
import functools

import jax
import jax.numpy as jnp
from jax import lax
from jax.experimental import pallas as pl
from jax.experimental.pallas import tpu as pltpu
from jax.experimental.pallas import tpu_sc as plsc

N = 10000
D = 128
E = 320000
NC = 2
NS = 16
NW = NC * NS
C = 128
E_PAD = 327680
EPW = E_PAD // NW
NCHUNKS = EPW // C
RPT = N // NS

_mesh = plsc.VectorSubcoreMesh(
    core_axis_name="c", subcore_axis_name="s", num_cores=NC, num_subcores=NS
)


@functools.partial(
    pl.kernel,
    out_type=jax.ShapeDtypeStruct((NC * N, D), jnp.float32),
    mesh=_mesh,
    scratch_types=[
        pltpu.VMEM((C,), jnp.int32),
        pltpu.VMEM((C,), jnp.int32),
        pltpu.VMEM((C,), jnp.float32),
        pltpu.VMEM((C, D), jnp.float32),
        pltpu.VMEM_SHARED((N, D), jnp.float32),
        pltpu.SemaphoreType.DMA,
    ],
)
def _spmm_sc(x_hbm, col_hbm, row_hbm, w_hbm, out_hbm, colv, rowv, wv, msg, acc,
             sem):
    c = lax.axis_index("c")
    s = lax.axis_index("s")
    wid = c * NS + s

    zeros16 = jnp.zeros((16,), jnp.float32)

    def _zrow(i, _):
        for j in range(D // 16):
            msg[i, pl.ds(j * 16, 16)] = zeros16
        return 0

    lax.fori_loop(0, C, _zrow, 0)

    row0 = s * RPT
    nfull = RPT // C
    for b in range(nfull):
        pltpu.sync_copy(msg, acc.at[pl.ds(row0 + b * C, C)])
    rem = RPT - nfull * C
    pltpu.sync_copy(msg.at[pl.ds(0, rem)],
                    acc.at[pl.ds(row0 + nfull * C, rem)])
    plsc.subcore_barrier()

    def _chunk(k, _):
        base = wid * EPW + k * C
        pltpu.sync_copy(col_hbm.at[pl.ds(base, C)], colv)
        pltpu.sync_copy(row_hbm.at[pl.ds(base, C)], rowv)
        pltpu.sync_copy(w_hbm.at[pl.ds(base, C)], wv)
        pltpu.async_copy(x_hbm.at[colv], msg, sem).wait()

        def _scale(i, _):
            wvec = plsc.load_gather(wv, [jnp.full((16,), i, jnp.int32)])
            for j in range(D // 16):
                sl = pl.ds(j * 16, 16)
                msg[i, sl] = msg[i, sl] * wvec
            return 0

        lax.fori_loop(0, C, _scale, 0)
        pltpu.sync_copy(msg, acc.at[rowv], add=True)
        return 0

    lax.fori_loop(0, NCHUNKS, _chunk, 0)
    plsc.subcore_barrier()

    pltpu.sync_copy(acc.at[pl.ds(row0, RPT)],
                    out_hbm.at[pl.ds(c * N + row0, RPT)])


def _add_body(a_ref, b_ref, o_ref):
    o_ref[...] = a_ref[...] + b_ref[...]


def _combine(partials):
    nb = 8
    rows = N // nb
    return pl.pallas_call(
        _add_body,
        out_shape=jax.ShapeDtypeStruct((N, D), jnp.float32),
        grid=(nb,),
        in_specs=[
            pl.BlockSpec((rows, D), lambda i: (i, 0)),
            pl.BlockSpec((rows, D), lambda i: (i + nb, 0)),
        ],
        out_specs=pl.BlockSpec((rows, D), lambda i: (i, 0)),
    )(partials, partials)


def kernel(x, edge_weight, edge_index):
    row = edge_index[0].astype(jnp.int32)
    col = edge_index[1].astype(jnp.int32)
    pad = E_PAD - E
    zi = jnp.zeros((pad,), jnp.int32)
    col = jnp.concatenate([col, zi])
    row = jnp.concatenate([row, zi])
    w = jnp.concatenate([edge_weight, jnp.zeros((pad,), jnp.float32)])
    partials = _spmm_sc(x, col, row, w)
    return _combine(partials)


# speedup vs baseline: 2.6940x; 2.6940x over previous
import functools

import jax
import jax.numpy as jnp
from jax import lax
from jax.experimental import pallas as pl
from jax.experimental.pallas import tpu as pltpu
from jax.experimental.pallas import tpu_sc as plsc

N = 10000
D = 128
E = 320000
NC = 2
NS = 16
NW = NC * NS
C = 128
E_PAD = 327680
EPW = E_PAD // NW
NCHUNKS = EPW // C
RPT = 632
NPAD = NS * RPT

_mesh = plsc.VectorSubcoreMesh(
    core_axis_name="c", subcore_axis_name="s", num_cores=NC, num_subcores=NS
)


@functools.partial(
    pl.kernel,
    out_type=jax.ShapeDtypeStruct((NC, NPAD, D), jnp.float32),
    mesh=_mesh,
    scratch_types=[
        pltpu.VMEM((C,), jnp.int32),
        pltpu.VMEM((C,), jnp.int32),
        pltpu.VMEM((C,), jnp.float32),
        pltpu.VMEM((C, D), jnp.float32),
        pltpu.VMEM_SHARED((NPAD, D), jnp.float32),
        pltpu.SemaphoreType.DMA,
    ],
)
def _spmm_sc(x_hbm, col_hbm, row_hbm, w_hbm, out_hbm, colv, rowv, wv, msg, acc,
             sem):
    c = lax.axis_index("c")
    s = lax.axis_index("s")
    wid = c * NS + s

    zeros16 = jnp.zeros((16,), jnp.float32)

    def _zrow(i, _):
        for j in range(D // 16):
            msg[i, pl.ds(j * 16, 16)] = zeros16
        return 0

    lax.fori_loop(0, C, _zrow, 0)

    row0 = s * RPT
    nfull = RPT // C
    for b in range(nfull):
        pltpu.sync_copy(msg, acc.at[pl.ds(row0 + b * C, C)])
    rem = RPT - nfull * C
    pltpu.sync_copy(msg.at[pl.ds(0, rem)],
                    acc.at[pl.ds(row0 + nfull * C, rem)])
    plsc.subcore_barrier()

    def _chunk(k, _):
        base = wid * EPW + k * C
        pltpu.sync_copy(col_hbm.at[pl.ds(base, C)], colv)
        pltpu.sync_copy(row_hbm.at[pl.ds(base, C)], rowv)
        pltpu.sync_copy(w_hbm.at[pl.ds(base, C)], wv)
        pltpu.async_copy(x_hbm.at[colv], msg, sem).wait()

        def _scale(g, _):
            w16 = wv[pl.ds(g * 16, 16)]
            for l in range(16):
                i = g * 16 + l
                ws = w16[l]
                for j in range(D // 16):
                    sl = pl.ds(j * 16, 16)
                    msg[i, sl] = msg[i, sl] * ws
            return 0

        lax.fori_loop(0, C // 16, _scale, 0)
        pltpu.sync_copy(msg, acc.at[rowv], add=True)
        return 0

    lax.fori_loop(0, NCHUNKS, _chunk, 0)
    plsc.subcore_barrier()

    pltpu.sync_copy(acc.at[pl.ds(row0, RPT)],
                    out_hbm.at[c].at[pl.ds(row0, RPT)])


def _add_body(a_ref, b_ref, o_ref):
    o_ref[...] = a_ref[0] + b_ref[0]


def _combine(partials):
    nb = 10
    rows = N // nb
    return pl.pallas_call(
        _add_body,
        out_shape=jax.ShapeDtypeStruct((N, D), jnp.float32),
        grid=(nb,),
        in_specs=[
            pl.BlockSpec((1, rows, D), lambda i: (0, i, 0)),
            pl.BlockSpec((1, rows, D), lambda i: (1, i, 0)),
        ],
        out_specs=pl.BlockSpec((rows, D), lambda i: (i, 0)),
    )(partials, partials)


def kernel(x, edge_weight, edge_index):
    row = edge_index[0].astype(jnp.int32)
    col = edge_index[1].astype(jnp.int32)
    pad = E_PAD - E
    zi = jnp.zeros((pad,), jnp.int32)
    col = jnp.concatenate([col, zi])
    row = jnp.concatenate([row, zi])
    w = jnp.concatenate([edge_weight, jnp.zeros((pad,), jnp.float32)])
    partials = _spmm_sc(x, col, row, w)
    return _combine(partials)


# speedup vs baseline: 3.2824x; 1.2184x over previous
import functools

import jax
import jax.numpy as jnp
from jax import lax
from jax.experimental import pallas as pl
from jax.experimental.pallas import tpu as pltpu
from jax.experimental.pallas import tpu_sc as plsc

N = 10000
D = 128
E = 320000
NC = 2
NS = 16
NW = NC * NS
C = 128
E_PAD = 327680
EPW = E_PAD // NW
NCHUNKS = EPW // C
NH = 2
CH = NCHUNKS // NH
NPAIR_H = CH // 2
RPT = 632

_mesh = plsc.VectorSubcoreMesh(
    core_axis_name="c", subcore_axis_name="s", num_cores=NC, num_subcores=NS
)


@functools.partial(
    pl.kernel,
    out_type=jax.ShapeDtypeStruct((NC, N, D), jnp.float32),
    mesh=_mesh,
    scratch_types=[
        pltpu.VMEM((CH, C), jnp.int32),
        pltpu.VMEM((CH, C), jnp.int32),
        pltpu.VMEM((CH, C), jnp.float32),
        pltpu.VMEM((C, D), jnp.float32),
        pltpu.VMEM((C, D), jnp.float32),
        pltpu.VMEM_SHARED((N, D), jnp.float32),
        pltpu.SemaphoreType.DMA,
        pltpu.SemaphoreType.DMA,
        pltpu.SemaphoreType.DMA,
        pltpu.SemaphoreType.DMA,
    ],
)
def _spmm_sc(x_hbm, col_hbm, row_hbm, w_hbm, out_hbm, colv, rowv, wv,
             msg0, msg1, acc, gsem0, gsem1, ssem0, ssem1):
    c = lax.axis_index("c")
    s = lax.axis_index("s")
    wid = c * NS + s

    zeros16 = jnp.zeros((16,), jnp.float32)

    def _zrow(i, _):
        for j in range(D // 16):
            msg0[i, pl.ds(j * 16, 16)] = zeros16
        return 0

    lax.fori_loop(0, C, _zrow, 0)

    row0 = s * RPT

    def _zero_acc(nrows):
        nfull = nrows // C
        for b in range(nfull):
            pltpu.sync_copy(msg0, acc.at[pl.ds(row0 + b * C, C)])
        rem = nrows - nfull * C
        if rem:
            pltpu.sync_copy(msg0.at[pl.ds(0, rem)],
                            acc.at[pl.ds(row0 + nfull * C, rem)])

    @pl.when(s < NS - 1)
    def _():
        _zero_acc(RPT)

    @pl.when(s == NS - 1)
    def _():
        _zero_acc(N - (NS - 1) * RPT)

    plsc.subcore_barrier()

    def _scale(msg, k):
        def body(g, _):
            w16 = wv[k, pl.ds(g * 16, 16)]
            for l in range(16):
                i = g * 16 + l
                ws = w16[l]
                for j in range(D // 16):
                    sl = pl.ds(j * 16, 16)
                    msg[i, sl] = msg[i, sl] * ws
            return 0

        lax.fori_loop(0, C // 16, body, 0)

    for h in range(NH):
        pltpu.sync_copy(col_hbm.at[wid].at[h], colv)
        pltpu.sync_copy(row_hbm.at[wid].at[h], rowv)
        pltpu.sync_copy(w_hbm.at[wid].at[h], wv)

        pltpu.async_copy(x_hbm.at[colv.at[0]], msg0, gsem0)
        pltpu.async_copy(x_hbm.at[colv.at[1]], msg1, gsem1)

        def _pair(t, _):
            k0 = 2 * t
            k1 = k0 + 1

            pltpu.make_async_copy(x_hbm.at[colv.at[k0]], msg0, gsem0).wait()
            _scale(msg0, k0)
            scat0 = pltpu.async_copy(msg0, acc.at[rowv.at[k0]], ssem0,
                                     add=True)

            pltpu.make_async_copy(x_hbm.at[colv.at[k1]], msg1, gsem1).wait()
            _scale(msg1, k1)
            scat1 = pltpu.async_copy(msg1, acc.at[rowv.at[k1]], ssem1,
                                     add=True)

            scat0.wait()
            scat1.wait()

            @pl.when(t + 1 < NPAIR_H)
            def _prefetch():
                pltpu.async_copy(x_hbm.at[colv.at[k0 + 2]], msg0, gsem0)
                pltpu.async_copy(x_hbm.at[colv.at[k1 + 2]], msg1, gsem1)

            return 0

        lax.fori_loop(0, NPAIR_H, _pair, 0)

    plsc.subcore_barrier()

    @pl.when(s < NS - 1)
    def _():
        pltpu.sync_copy(acc.at[pl.ds(row0, RPT)],
                        out_hbm.at[c].at[pl.ds(row0, RPT)])

    @pl.when(s == NS - 1)
    def _():
        last = N - (NS - 1) * RPT
        pltpu.sync_copy(acc.at[pl.ds(row0, last)],
                        out_hbm.at[c].at[pl.ds(row0, last)])


def _add_body(a_ref, b_ref, o_ref):
    o_ref[...] = a_ref[0] + b_ref[0]


def _combine(partials):
    nb = 10
    rows = N // nb
    return pl.pallas_call(
        _add_body,
        out_shape=jax.ShapeDtypeStruct((N, D), jnp.float32),
        grid=(nb,),
        in_specs=[
            pl.BlockSpec((1, rows, D), lambda i: (0, i, 0)),
            pl.BlockSpec((1, rows, D), lambda i: (1, i, 0)),
        ],
        out_specs=pl.BlockSpec((rows, D), lambda i: (i, 0)),
    )(partials, partials)


def kernel(x, edge_weight, edge_index):
    row = edge_index[0].astype(jnp.int32)
    col = edge_index[1].astype(jnp.int32)
    pad = E_PAD - E
    zi = jnp.zeros((pad,), jnp.int32)
    col = jnp.concatenate([col, zi]).reshape(NW, NH, CH, C)
    row = jnp.concatenate([row, zi]).reshape(NW, NH, CH, C)
    w = jnp.concatenate([edge_weight, jnp.zeros((pad,), jnp.float32)])
    w = w.reshape(NW, NH, CH, C)
    partials = _spmm_sc(x, col, row, w)
    return _combine(partials)


# speedup vs baseline: 3.9158x; 1.1930x over previous
import functools

import jax
import jax.numpy as jnp
from jax import lax
from jax.experimental import pallas as pl
from jax.experimental.pallas import tpu as pltpu
from jax.experimental.pallas import tpu_sc as plsc

N = 10000
D = 128
DH = D // 2
E = 320000
NC = 2
NS = 16
C = 128
E_PAD = 327680
EPW = E_PAD // NS
NCHUNKS = EPW // C
NQ = 4
CQ = NCHUNKS // NQ
NPAIR_Q = CQ // 2
RPT = 632

_mesh = plsc.VectorSubcoreMesh(
    core_axis_name="c", subcore_axis_name="s", num_cores=NC, num_subcores=NS
)


@functools.partial(
    pl.kernel,
    out_type=jax.ShapeDtypeStruct((NC, N, DH), jnp.float32),
    mesh=_mesh,
    compiler_params=pltpu.CompilerParams(use_tc_tiling_on_sc=False),
    scratch_types=[
        pltpu.VMEM((CQ, C), jnp.int32),
        pltpu.VMEM((CQ, C), jnp.int32),
        pltpu.VMEM((CQ, C), jnp.float32),
        pltpu.VMEM((C, DH), jnp.float32),
        pltpu.VMEM((C, DH), jnp.float32),
        pltpu.VMEM_SHARED((N, DH), jnp.float32),
        pltpu.VMEM_SHARED((N, DH), jnp.float32),
        pltpu.SemaphoreType.DMA,
        pltpu.SemaphoreType.DMA,
        pltpu.SemaphoreType.DMA,
        pltpu.SemaphoreType.DMA,
    ],
)
def _spmm_sc(x_hbm, col_hbm, row_hbm, w_hbm, out_hbm, colv, rowv, wv,
             msg0, msg1, xs, acc, gsem0, gsem1, ssem0, ssem1):
    c = lax.axis_index("c")
    s = lax.axis_index("s")

    zeros16 = jnp.zeros((16,), jnp.float32)

    def _zrow(i, _):
        for j in range(DH // 16):
            msg0[i, pl.ds(j * 16, 16)] = zeros16
        return 0

    lax.fori_loop(0, C, _zrow, 0)

    row0 = s * RPT

    def _init_rows(nrows):
        pltpu.sync_copy(x_hbm.at[c].at[pl.ds(row0, nrows)],
                        xs.at[pl.ds(row0, nrows)])
        nfull = nrows // C
        for b in range(nfull):
            pltpu.sync_copy(msg0, acc.at[pl.ds(row0 + b * C, C)])
        rem = nrows - nfull * C
        if rem:
            pltpu.sync_copy(msg0.at[pl.ds(0, rem)],
                            acc.at[pl.ds(row0 + nfull * C, rem)])

    @pl.when(s < NS - 1)
    def _():
        _init_rows(RPT)

    @pl.when(s == NS - 1)
    def _():
        _init_rows(N - (NS - 1) * RPT)

    plsc.subcore_barrier()

    def _scale(msg, k):
        def body(g, _):
            w16 = wv[k, pl.ds(g * 16, 16)]
            for l in range(16):
                i = g * 16 + l
                ws = w16[l]
                for j in range(DH // 16):
                    sl = pl.ds(j * 16, 16)
                    msg[i, sl] = msg[i, sl] * ws
            return 0

        lax.fori_loop(0, C // 16, body, 0)

    for q in range(NQ):
        pltpu.sync_copy(col_hbm.at[s].at[q], colv)
        pltpu.sync_copy(row_hbm.at[s].at[q], rowv)
        pltpu.sync_copy(w_hbm.at[s].at[q], wv)

        pltpu.async_copy(xs.at[colv.at[0]], msg0, gsem0)
        pltpu.async_copy(xs.at[colv.at[1]], msg1, gsem1)

        def _pair(t, _):
            k0 = 2 * t
            k1 = k0 + 1

            pltpu.make_async_copy(xs.at[colv.at[k0]], msg0, gsem0).wait()
            _scale(msg0, k0)
            scat0 = pltpu.async_copy(msg0, acc.at[rowv.at[k0]], ssem0,
                                     add=True)

            pltpu.make_async_copy(xs.at[colv.at[k1]], msg1, gsem1).wait()
            _scale(msg1, k1)
            scat1 = pltpu.async_copy(msg1, acc.at[rowv.at[k1]], ssem1,
                                     add=True)

            scat0.wait()
            scat1.wait()

            @pl.when(t + 1 < NPAIR_Q)
            def _prefetch():
                pltpu.async_copy(xs.at[colv.at[k0 + 2]], msg0, gsem0)
                pltpu.async_copy(xs.at[colv.at[k1 + 2]], msg1, gsem1)

            return 0

        lax.fori_loop(0, NPAIR_Q, _pair, 0)

    plsc.subcore_barrier()

    @pl.when(s < NS - 1)
    def _():
        pltpu.sync_copy(acc.at[pl.ds(row0, RPT)],
                        out_hbm.at[c].at[pl.ds(row0, RPT)])

    @pl.when(s == NS - 1)
    def _():
        last = N - (NS - 1) * RPT
        pltpu.sync_copy(acc.at[pl.ds(row0, last)],
                        out_hbm.at[c].at[pl.ds(row0, last)])


def kernel(x, edge_weight, edge_index):
    row = edge_index[0].astype(jnp.int32)
    col = edge_index[1].astype(jnp.int32)
    pad = E_PAD - E
    zi = jnp.zeros((pad,), jnp.int32)
    col = jnp.concatenate([col, zi]).reshape(NS, NQ, CQ, C)
    row = jnp.concatenate([row, zi]).reshape(NS, NQ, CQ, C)
    w = jnp.concatenate([edge_weight, jnp.zeros((pad,), jnp.float32)])
    w = w.reshape(NS, NQ, CQ, C)
    xh = x.reshape(N, NC, DH).transpose(1, 0, 2)
    partials = _spmm_sc(xh, col, row, w)
    return partials.transpose(1, 0, 2).reshape(N, D)
